# Initial kernel scaffold; baseline (speedup 1.0000x reference)
import jax
import jax.numpy as jnp
from jax.experimental import pallas as pl


def kernel(memory, node_idxs, values):
    raise NotImplementedError("write your pallas kernel here")


# speedup vs baseline: 4.3235x; 4.3235x over previous
import functools

import jax
import jax.numpy as jnp
from jax import lax
from jax.experimental import pallas as pl
from jax.experimental.pallas import tpu as pltpu

N_NODES = 100000
D = 129
B = 4096
BLK = 1000
GRID = N_NODES // BLK


def _body(sidx_s, perm_s, mem_ref, vals_ref, sidx_v_ref, out_ref):
    i = pl.program_id(0)
    out_ref[...] = mem_ref[...]
    lo = i * BLK
    sidx_v = sidx_v_ref[...]
    cnt_lo = jnp.sum((sidx_v < lo).astype(jnp.int32))
    cnt_hi = jnp.sum((sidx_v < lo + BLK).astype(jnp.int32))

    def apply_one(k, carry):
        row = sidx_s[k] - lo
        src = perm_s[k]
        out_ref[pl.ds(row, 1), :] = vals_ref[pl.ds(src, 1), :]
        return carry

    lax.fori_loop(cnt_lo, cnt_hi, apply_one, 0)


_call = pl.pallas_call(
    _body,
    grid_spec=pltpu.PrefetchScalarGridSpec(
        num_scalar_prefetch=2,
        grid=(GRID,),
        in_specs=[
            pl.BlockSpec((BLK, D), lambda i, *_: (i, 0)),
            pl.BlockSpec((B, D), lambda i, *_: (0, 0)),
            pl.BlockSpec((B,), lambda i, *_: (0,)),
        ],
        out_specs=pl.BlockSpec((BLK, D), lambda i, *_: (i, 0)),
    ),
    out_shape=jax.ShapeDtypeStruct((N_NODES, D), jnp.float32),
)


def kernel(memory, node_idxs, values):
    idx = node_idxs.astype(jnp.int32)
    perm = jnp.argsort(idx, stable=True).astype(jnp.int32)
    sidx = idx[perm]
    return _call(sidx, perm, memory, values, sidx)


# speedup vs baseline: 5.1185x; 1.1839x over previous
import functools

import jax
import jax.numpy as jnp
from jax import lax
from jax.experimental import pallas as pl
from jax.experimental.pallas import tpu as pltpu

N_NODES = 100000
D = 129
B = 4096
BLK = 4000
GRID = N_NODES // BLK


def _body(sidx_s, perm_s, mem_ref, vals_ref, sidx_v_ref, out_ref):
    i = pl.program_id(0)
    out_ref[...] = mem_ref[...]
    lo = i * BLK
    sidx_v = sidx_v_ref[...]
    cnt_lo = jnp.sum((sidx_v < lo).astype(jnp.int32))
    cnt_hi = jnp.sum((sidx_v < lo + BLK).astype(jnp.int32))

    def apply_one(k, carry):
        row = sidx_s[k] - lo
        src = perm_s[k]
        out_ref[pl.ds(row, 1), :] = vals_ref[pl.ds(src, 1), :]
        return carry

    lax.fori_loop(cnt_lo, cnt_hi, apply_one, 0)


_call = pl.pallas_call(
    _body,
    grid_spec=pltpu.PrefetchScalarGridSpec(
        num_scalar_prefetch=2,
        grid=(GRID,),
        in_specs=[
            pl.BlockSpec((BLK, D), lambda i, *_: (i, 0)),
            pl.BlockSpec((B, D), lambda i, *_: (0, 0)),
            pl.BlockSpec((B,), lambda i, *_: (0,)),
        ],
        out_specs=pl.BlockSpec((BLK, D), lambda i, *_: (i, 0)),
    ),
    out_shape=jax.ShapeDtypeStruct((N_NODES, D), jnp.float32),
)


def kernel(memory, node_idxs, values):
    idx = node_idxs.astype(jnp.int32)
    perm = jnp.argsort(idx, stable=True).astype(jnp.int32)
    sidx = idx[perm]
    return _call(sidx, perm, memory, values, sidx)
